# Initial kernel scaffold; baseline (speedup 1.0000x reference)
import jax
import jax.numpy as jnp
from jax.experimental import pallas as pl


def kernel(img, edges):
    raise NotImplementedError("write your pallas kernel here")


# speedup vs baseline: 10.0223x; 10.0223x over previous
import jax
import jax.numpy as jnp
from jax import lax
from jax.experimental import pallas as pl
from jax.experimental.pallas import tpu as pltpu

_F32 = jnp.float32
_I32 = jnp.int32

_FIXED = 196
_NSPLIT = (_FIXED - 1) // 3
_PAD = 512


def _sel(n, m, shift, scale):
    i = lax.broadcasted_iota(_I32, (n, m), 0)
    j = lax.broadcasted_iota(_I32, (n, m), 1)
    return jnp.where((j >> shift) == i, scale, 0.0).astype(_F32)


def _selT(m, n, shift, scale):
    j = lax.broadcasted_iota(_I32, (m, n), 0)
    i = lax.broadcasted_iota(_I32, (m, n), 1)
    return jnp.where((j >> shift) == i, scale, 0.0).astype(_F32)


def _dot(a, b):
    return jnp.dot(a, b, preferred_element_type=_F32,
                   precision=lax.Precision.HIGHEST)


def _kernel_body(img_ref, edges_ref, out_ref, ep_ref, ip_ref):
    A = _sel(32, 512, 4, 1.0)
    At = _selT(512, 32, 4, 1.0)
    cur = _dot(_dot(A, edges_ref[...]), At)
    ep_ref[0:32, 0:32] = cur
    for L in range(1, 6):
        n = 32 >> L
        B = _sel(n, 2 * n, 1, 1.0)
        Bt = _selT(2 * n, n, 1, 1.0)
        cur = _dot(_dot(B, cur), Bt)
        off = 64 - (64 >> L)
        ep_ref[off:off + n, 0:n] = cur
    root_mass = cur[0, 0]

    ip_ref[:, 512:1024, :] = jnp.zeros((3, 512, 512), _F32)
    ip_ref[:, 0:512, :] = img_ref[...]
    for c in range(3):
        curi = img_ref[c]
        for L in range(1, 6):
            n = 512 >> L
            B = _sel(n, 2 * n, 1, 0.5)
            Bt = _selT(2 * n, n, 1, 0.5)
            curi = _dot(_dot(B, curi), Bt)
            off = 1024 - (1024 >> L)
            ip_ref[c, off:off + n, 0:n] = curi

    idx = lax.broadcasted_iota(_I32, (1, _PAD), 1)
    col32 = lax.broadcasted_iota(_I32, (1, 32), 1)

    sub8 = lax.broadcasted_iota(_I32, (8, 32), 0)
    lane8 = lax.broadcasted_iota(_I32, (8, 32), 1)

    def child_mass(cy, cx, lc):
        row = (64 - jnp.right_shift(jnp.int32(64), lc)
               + jnp.right_shift(cy, lc + 4))
        colv = jnp.right_shift(cx, lc + 4)
        rowa = jnp.right_shift(row, 3) * 8
        blk = ep_ref[pl.ds(rowa, 8), :]
        return jnp.sum(jnp.where((sub8 == row - rowa) & (lane8 == colv),
                                 blk, 0.0))

    ys0 = jnp.zeros((1, _PAD), _I32)
    xs0 = jnp.zeros((1, _PAD), _I32)
    ls0 = jnp.where(idx == 0, 5, 0).astype(_I32)
    ms0 = jnp.where(idx == 0, root_mass, 0.0).astype(_F32)
    ac0 = jnp.where(idx == 0, 1, 0).astype(_I32)

    def split_body(t, st):
        ys, xs, ls, ms, ac = st
        cand = jnp.where((ac == 1) & (ls >= 1), ms, -jnp.inf)
        mx = jnp.max(cand)
        b = jnp.min(jnp.where(cand == mx, idx, jnp.int32(1 << 20)))
        yb = jnp.sum(jnp.where(idx == b, ys, 0))
        xb = jnp.sum(jnp.where(idx == b, xs, 0))
        lb = jnp.sum(jnp.where(idx == b, ls, 0))
        h = jnp.left_shift(jnp.int32(8), lb)
        lc = lb - 1
        base = 1 + 4 * t
        ac = jnp.where(idx == b, 0, ac)
        cys = (yb, yb, yb + h, yb + h)
        cxs = (xb, xb + h, xb, xb + h)
        for k in range(4):
            mk = child_mass(cys[k], cxs[k], lc)
            sel = idx == (base + k)
            ys = jnp.where(sel, cys[k], ys)
            xs = jnp.where(sel, cxs[k], xs)
            ls = jnp.where(sel, lc, ls)
            ms = jnp.where(sel, mk, ms)
            ac = jnp.where(sel, 1, ac)
        return ys, xs, ls, ms, ac

    ys, xs, ls, ms, ac = lax.fori_loop(
        0, _NSPLIT, split_body, (ys0, xs0, ls0, ms0, ac0))

    y5 = jnp.right_shift(ys, 4)
    x5 = jnp.right_shift(xs, 4)
    key = jnp.zeros((1, _PAD), _I32)
    for b in range(5):
        key = key | (((y5 >> b) & 1) << (2 * b + 1)) | (((x5 >> b) & 1) << (2 * b))

    jj = lax.broadcasted_iota(_I32, (512, 16), 0)
    kk = lax.broadcasted_iota(_I32, (512, 16), 1)

    def gather_body(p, rem):
        keym = jnp.where(rem == 1, key, jnp.int32(1 << 20))
        mn = jnp.min(keym)
        b = jnp.min(jnp.where(keym == mn, idx, jnp.int32(1 << 20)))
        yb = jnp.sum(jnp.where(idx == b, ys, 0))
        xb = jnp.sum(jnp.where(idx == b, xs, 0))
        lb = jnp.sum(jnp.where(idx == b, ls, 0))
        rem = jnp.where(idx == b, 0, rem)
        row = (1024 - jnp.right_shift(jnp.int32(1024), lb)
               + jnp.right_shift(yb, lb))
        row = jnp.right_shift(row, 3) * 8
        colv = jnp.right_shift(xb, lb)
        rows16 = ip_ref[:, pl.ds(row, 16), :]
        smat = jnp.where(jj == colv + kk, 1.0, 0.0).astype(_F32)
        patch = lax.dot_general(rows16, smat, (((2,), (0,)), ((), ())),
                                preferred_element_type=_F32,
                                precision=lax.Precision.HIGHEST)
        out_ref[:, pl.ds(p, 1), :, :] = patch[:, None, :, :]
        return rem

    lax.fori_loop(0, _FIXED, gather_body, ac)


def kernel(img, edges):
    patches = pl.pallas_call(
        _kernel_body,
        out_shape=jax.ShapeDtypeStruct((3, _FIXED, 16, 16), _F32),
        scratch_shapes=[
            pltpu.VMEM((64, 32), _F32),
            pltpu.VMEM((3, 1024, 512), _F32),
        ],
    )(img, edges)
    return jnp.reshape(jnp.transpose(patches, (1, 2, 3, 0)), (3, _FIXED, 256))


# speedup vs baseline: 12.0798x; 1.2053x over previous
import jax
import jax.numpy as jnp
from jax import lax
from jax.experimental import pallas as pl
from jax.experimental.pallas import tpu as pltpu

_F32 = jnp.float32
_I32 = jnp.int32

_FIXED = 196
_NSPLIT = (_FIXED - 1) // 3
_PAD = 512


def _sel(n, m, shift, scale):
    i = lax.broadcasted_iota(_I32, (n, m), 0)
    j = lax.broadcasted_iota(_I32, (n, m), 1)
    return jnp.where((j >> shift) == i, scale, 0.0).astype(_F32)


def _selT(m, n, shift, scale):
    j = lax.broadcasted_iota(_I32, (m, n), 0)
    i = lax.broadcasted_iota(_I32, (m, n), 1)
    return jnp.where((j >> shift) == i, scale, 0.0).astype(_F32)


def _dot(a, b):
    return jnp.dot(a, b, preferred_element_type=_F32,
                   precision=lax.Precision.HIGHEST)


def _kernel_body(img_ref, edges_ref, out_ref, ep_ref, ip_ref):
    A = _sel(32, 512, 4, 1.0)
    At = _selT(512, 32, 4, 1.0)
    cur = _dot(_dot(A, edges_ref[...]), At)
    ep_ref[0:32, 0:32] = cur
    for L in range(1, 6):
        n = 32 >> L
        B = _sel(n, 2 * n, 1, 1.0)
        Bt = _selT(2 * n, n, 1, 1.0)
        cur = _dot(_dot(B, cur), Bt)
        off = 64 - (64 >> L)
        ep_ref[off:off + n, 0:n] = cur
    root_mass = cur[0, 0]

    ip_ref[:, 512:1024, :] = jnp.zeros((3, 512, 512), _F32)
    ip_ref[:, 0:512, :] = img_ref[...]
    for c in range(3):
        curi = img_ref[c]
        for L in range(1, 6):
            n = 512 >> L
            B = _sel(n, 2 * n, 1, 0.5)
            Bt = _selT(2 * n, n, 1, 0.5)
            curi = _dot(_dot(B, curi), Bt)
            off = 1024 - (1024 >> L)
            ip_ref[c, off:off + n, 0:n] = curi

    idx = lax.broadcasted_iota(_I32, (1, _PAD), 1)
    col32 = lax.broadcasted_iota(_I32, (1, 32), 1)

    sub8 = lax.broadcasted_iota(_I32, (8, 32), 0)
    lane8 = lax.broadcasted_iota(_I32, (8, 32), 1)

    def child_mass(cy, cx, lc):
        row = (64 - jnp.right_shift(jnp.int32(64), lc)
               + jnp.right_shift(cy, lc + 4))
        colv = jnp.right_shift(cx, lc + 4)
        rowa = jnp.right_shift(row, 3) * 8
        blk = ep_ref[pl.ds(rowa, 8), :]
        return jnp.sum(jnp.where((sub8 == row - rowa) & (lane8 == colv),
                                 blk, 0.0))

    codes0 = jnp.where(idx == 0, 5, 0).astype(_I32)
    ms0 = jnp.where(idx == 0, root_mass, 0.0).astype(_F32)
    ac0 = jnp.where(idx == 0, 1, 0).astype(_I32)

    def split_body(t, st):
        codes, ms, ac = st
        cand = jnp.where((ac == 1) & ((codes & 7) >= 1), ms, -jnp.inf)
        mx = jnp.max(cand)
        b = jnp.min(jnp.where(cand == mx, idx, jnp.int32(1 << 20)))
        cb = jnp.sum(jnp.where(idx == b, codes, 0))
        yb = cb >> 12
        xb = (cb >> 3) & 511
        lb = cb & 7
        h = jnp.left_shift(jnp.int32(8), lb)
        lc = lb - 1
        base = 1 + 4 * t
        ac = jnp.where(idx == b, 0, ac)
        cys = (yb, yb, yb + h, yb + h)
        cxs = (xb, xb + h, xb, xb + h)
        for k in range(4):
            mk = child_mass(cys[k], cxs[k], lc)
            sel = idx == (base + k)
            codes = jnp.where(sel, (cys[k] << 12) | (cxs[k] << 3) | lc, codes)
            ms = jnp.where(sel, mk, ms)
            ac = jnp.where(sel, 1, ac)
        return codes, ms, ac

    codes, ms, ac = lax.fori_loop(0, _NSPLIT, split_body, (codes0, ms0, ac0))

    y5 = codes >> 16
    x5 = (codes >> 7) & 31
    key = jnp.zeros((1, _PAD), _I32)
    for b in range(5):
        key = key | (((y5 >> b) & 1) << (2 * b + 1)) | (((x5 >> b) & 1) << (2 * b))

    sq_i = lax.broadcasted_iota(_I32, (_PAD, _PAD), 0)
    sq_j = lax.broadcasted_iota(_I32, (_PAD, _PAD), 1)
    diag = (sq_i == sq_j)
    ones_col = jnp.full((_PAD, 1), 1.0, _F32)

    def _transp(v):
        vb = jnp.broadcast_to(v.astype(_F32), (_PAD, _PAD))
        return _dot(jnp.where(diag, vb, 0.0), ones_col)

    keyf = key.astype(_F32)
    actf = (ac == 1)
    key_s = jnp.broadcast_to(_transp(key), (_PAD, _PAD))
    key_l = jnp.broadcast_to(keyf, (_PAD, _PAD))
    act_l = jnp.broadcast_to(actf, (_PAD, _PAD))
    cnt = jnp.where((key_l < key_s) & act_l, 1.0, 0.0)
    rank_col = jnp.sum(cnt, axis=1, keepdims=True)
    rank_s = jnp.broadcast_to(rank_col, (_PAD, _PAD))
    rank_l = _dot(jnp.full((1, _PAD), 1.0, _F32),
                  jnp.where(diag, rank_s, 0.0))
    pm = jnp.where((sq_i.astype(_F32) == jnp.broadcast_to(rank_l, (_PAD, _PAD)))
                   & act_l, 1.0, 0.0)
    ordered = _dot(pm, _transp(codes))

    jj = lax.broadcasted_iota(_I32, (512, 16), 0)
    kk = lax.broadcasted_iota(_I32, (512, 16), 1)
    subcol = lax.broadcasted_iota(_I32, (_PAD, 1), 0)

    def gather_body(p, carry):
        code = jnp.sum(jnp.where(subcol == p, ordered, 0.0)).astype(_I32)
        yb = code >> 12
        xb = (code >> 3) & 511
        lb = code & 7
        row = (1024 - jnp.right_shift(jnp.int32(1024), lb)
               + jnp.right_shift(yb, lb))
        row = jnp.right_shift(row, 3) * 8
        colv = jnp.right_shift(xb, lb)
        rows16 = ip_ref[:, pl.ds(row, 16), :]
        smat = jnp.where(jj == colv + kk, 1.0, 0.0).astype(_F32)
        patch = lax.dot_general(rows16, smat, (((2,), (0,)), ((), ())),
                                preferred_element_type=_F32,
                                precision=lax.Precision.HIGHEST)
        out_ref[:, pl.ds(p, 1), :, :] = patch[:, None, :, :]
        return carry

    lax.fori_loop(0, _FIXED, gather_body, 0)


def kernel(img, edges):
    patches = pl.pallas_call(
        _kernel_body,
        out_shape=jax.ShapeDtypeStruct((3, _FIXED, 16, 16), _F32),
        scratch_shapes=[
            pltpu.VMEM((64, 32), _F32),
            pltpu.VMEM((3, 1024, 512), _F32),
        ],
    )(img, edges)
    return jnp.reshape(jnp.transpose(patches, (1, 2, 3, 0)), (3, _FIXED, 256))


# speedup vs baseline: 17.0823x; 1.4141x over previous
import jax
import jax.numpy as jnp
from jax import lax
from jax.experimental import pallas as pl
from jax.experimental.pallas import tpu as pltpu

_F32 = jnp.float32
_I32 = jnp.int32

_FIXED = 196
_NSPLIT = (_FIXED - 1) // 3
_PAD = 512


def _sel(n, m, shift, scale):
    i = lax.broadcasted_iota(_I32, (n, m), 0)
    j = lax.broadcasted_iota(_I32, (n, m), 1)
    return jnp.where((j >> shift) == i, scale, 0.0).astype(_F32)


def _selT(m, n, shift, scale):
    j = lax.broadcasted_iota(_I32, (m, n), 0)
    i = lax.broadcasted_iota(_I32, (m, n), 1)
    return jnp.where((j >> shift) == i, scale, 0.0).astype(_F32)


def _dot(a, b):
    return jnp.dot(a, b, preferred_element_type=_F32,
                   precision=lax.Precision.HIGHEST)


def _kernel_body(img_ref, edges_ref, out_ref, ep_ref, ip_ref):
    A = _sel(32, 512, 4, 1.0)
    At = _selT(512, 32, 4, 1.0)
    cur = _dot(_dot(A, edges_ref[...]), At)
    ep_ref[0:32, 0:32] = cur
    for L in range(1, 6):
        n = 32 >> L
        B = _sel(n, 2 * n, 1, 1.0)
        Bt = _selT(2 * n, n, 1, 1.0)
        cur = _dot(_dot(B, cur), Bt)
        off = 64 - (64 >> L)
        ep_ref[off:off + n, 0:n] = cur
    root_mass = cur[0, 0]

    ip_ref[:, 512:1024, :] = jnp.zeros((3, 512, 512), _F32)
    ip_ref[:, 0:512, :] = img_ref[...]
    for c in range(3):
        curi = img_ref[c]
        for L in range(1, 6):
            n = 512 >> L
            B = _sel(n, 2 * n, 1, 0.5)
            Bt = _selT(2 * n, n, 1, 0.5)
            curi = _dot(_dot(B, curi), Bt)
            off = 1024 - (1024 >> L)
            ip_ref[c, off:off + n, 0:n] = curi

    idx = lax.broadcasted_iota(_I32, (1, _PAD), 1)
    col32 = lax.broadcasted_iota(_I32, (1, 32), 1)

    sub8 = lax.broadcasted_iota(_I32, (8, 32), 0)
    lane8 = lax.broadcasted_iota(_I32, (8, 32), 1)

    codes0 = jnp.where(idx == 0, 5, 0).astype(_I32)
    ms0 = jnp.where(idx == 0, root_mass, 0.0).astype(_F32)
    ac0 = jnp.where(idx == 0, 1, 0).astype(_I32)

    def split_body(t, st):
        codes, ms, ac = st
        cand = jnp.where((ac == 1) & ((codes & 7) >= 1), ms, -jnp.inf)
        mx = jnp.max(cand)
        b = jnp.min(jnp.where(cand == mx, idx, jnp.int32(1 << 20)))
        cb = jnp.sum(jnp.where(idx == b, codes, 0))
        yb = cb >> 12
        xb = (cb >> 3) & 511
        lb = cb & 7
        h = jnp.left_shift(jnp.int32(8), lb)
        lc = lb - 1
        base = 1 + 4 * t
        ac = jnp.where(idx == b, 0, ac)
        cys = (yb, yb, yb + h, yb + h)
        cxs = (xb, xb + h, xb, xb + h)
        ri = jnp.right_shift(yb, lc + 4)
        ci = jnp.right_shift(xb, lc + 4)
        row = 64 - jnp.right_shift(jnp.int32(64), lc) + ri
        rowa = jnp.right_shift(row, 3) * 8
        sr = row - rowa
        blk = ep_ref[pl.ds(rowa, 8), :]
        mks = []
        for k in range(4):
            mask = (sub8 == sr + (k >> 1)) & (lane8 == ci + (k & 1))
            mks.append(jnp.sum(jnp.where(mask, blk, 0.0)))
        for k in range(4):
            sel = idx == (base + k)
            codes = jnp.where(sel, (cys[k] << 12) | (cxs[k] << 3) | lc, codes)
            ms = jnp.where(sel, mks[k], ms)
            ac = jnp.where(sel, 1, ac)
        return codes, ms, ac

    codes, ms, ac = lax.fori_loop(0, _NSPLIT, split_body, (codes0, ms0, ac0))

    y5 = codes >> 16
    x5 = (codes >> 7) & 31
    key = jnp.zeros((1, _PAD), _I32)
    for b in range(5):
        key = key | (((y5 >> b) & 1) << (2 * b + 1)) | (((x5 >> b) & 1) << (2 * b))

    sq_i = lax.broadcasted_iota(_I32, (_PAD, _PAD), 0)
    sq_j = lax.broadcasted_iota(_I32, (_PAD, _PAD), 1)
    diag = (sq_i == sq_j)
    ones_col = jnp.full((_PAD, 1), 1.0, _F32)

    def _transp(v):
        vb = jnp.broadcast_to(v.astype(_F32), (_PAD, _PAD))
        return _dot(jnp.where(diag, vb, 0.0), ones_col)

    keyf = key.astype(_F32)
    actf = (ac == 1)
    key_s = jnp.broadcast_to(_transp(key), (_PAD, _PAD))
    key_l = jnp.broadcast_to(keyf, (_PAD, _PAD))
    act_l = jnp.broadcast_to(actf, (_PAD, _PAD))
    cnt = jnp.where((key_l < key_s) & act_l, 1.0, 0.0)
    rank_col = jnp.sum(cnt, axis=1, keepdims=True)
    rank_s = jnp.broadcast_to(rank_col, (_PAD, _PAD))
    rank_l = _dot(jnp.full((1, _PAD), 1.0, _F32),
                  jnp.where(diag, rank_s, 0.0))
    pm = jnp.where((sq_i.astype(_F32) == jnp.broadcast_to(rank_l, (_PAD, _PAD)))
                   & act_l, 1.0, 0.0)
    ordered = _dot(pm, _transp(codes))

    subcol = lax.broadcasted_iota(_I32, (_PAD, 1), 0)

    def gather_one(p):
        code = jnp.sum(jnp.where(subcol == p, ordered, 0.0)).astype(_I32)
        yb = code >> 12
        xb = (code >> 3) & 511
        lb = code & 7
        row = (1024 - jnp.right_shift(jnp.int32(1024), lb)
               + jnp.right_shift(yb, lb))
        row = jnp.right_shift(row, 3) * 8
        colv = jnp.right_shift(xb, lb)
        rows16 = ip_ref[:, pl.ds(row, 16), :]
        patch = pltpu.roll(rows16, -colv, 2)[:, :, 0:16]
        out_ref[:, pl.ds(p, 1), :, :] = patch[:, None, :, :]

    def gather_body(i, carry):
        gather_one(2 * i)
        gather_one(2 * i + 1)
        return carry

    lax.fori_loop(0, _FIXED // 2, gather_body, 0)


def kernel(img, edges):
    patches = pl.pallas_call(
        _kernel_body,
        out_shape=jax.ShapeDtypeStruct((3, _FIXED, 16, 16), _F32),
        scratch_shapes=[
            pltpu.VMEM((64, 32), _F32),
            pltpu.VMEM((3, 1024, 512), _F32),
        ],
    )(img, edges)
    return jnp.reshape(jnp.transpose(patches, (1, 2, 3, 0)), (3, _FIXED, 256))


# speedup vs baseline: 18.5433x; 1.0855x over previous
import jax
import jax.numpy as jnp
from jax import lax
from jax.experimental import pallas as pl
from jax.experimental.pallas import tpu as pltpu

_F32 = jnp.float32
_I32 = jnp.int32

_FIXED = 196
_NSPLIT = (_FIXED - 1) // 3
_PAD = 512


def _sel(n, m, shift, scale):
    i = lax.broadcasted_iota(_I32, (n, m), 0)
    j = lax.broadcasted_iota(_I32, (n, m), 1)
    return jnp.where((j >> shift) == i, scale, 0.0).astype(_F32)


def _selT(m, n, shift, scale):
    j = lax.broadcasted_iota(_I32, (m, n), 0)
    i = lax.broadcasted_iota(_I32, (m, n), 1)
    return jnp.where((j >> shift) == i, scale, 0.0).astype(_F32)


def _dot(a, b):
    return jnp.dot(a, b, preferred_element_type=_F32,
                   precision=lax.Precision.HIGHEST)


def _kernel_body(img_ref, edges_ref, out_ref, ep_ref, ip_ref):
    A = _sel(32, 512, 4, 1.0)
    At = _selT(512, 32, 4, 1.0)
    cur = _dot(_dot(A, edges_ref[...]), At)
    ep_ref[0:32, 0:32] = cur
    for L in range(1, 6):
        n = 32 >> L
        B = _sel(n, 2 * n, 1, 1.0)
        Bt = _selT(2 * n, n, 1, 1.0)
        cur = _dot(_dot(B, cur), Bt)
        off = 64 - (64 >> L)
        ep_ref[off:off + n, 0:n] = cur
    root_mass = cur[0, 0]

    ip_ref[:, 512:1024, :] = jnp.zeros((3, 512, 512), _F32)
    ip_ref[:, 0:512, :] = img_ref[...]
    for c in range(3):
        curi = img_ref[c]
        for L in range(1, 6):
            n = 512 >> L
            B = _sel(n, 2 * n, 1, 0.5)
            Bt = _selT(2 * n, n, 1, 0.5)
            curi = _dot(_dot(B, curi), Bt)
            off = 1024 - (1024 >> L)
            ip_ref[c, off:off + n, 0:n] = curi

    idx = lax.broadcasted_iota(_I32, (1, _PAD), 1)
    col32 = lax.broadcasted_iota(_I32, (1, 32), 1)

    sub8 = lax.broadcasted_iota(_I32, (8, 32), 0)
    lane8 = lax.broadcasted_iota(_I32, (8, 32), 1)

    codes0 = jnp.where(idx == 0, 5, 0).astype(_I32)
    ms0 = jnp.where(idx == 0, root_mass, 0.0).astype(_F32)
    ac0 = jnp.where(idx == 0, 1, 0).astype(_I32)

    def split_body(t, st):
        codes, ms, ac = st
        cand = jnp.where((ac == 1) & ((codes & 7) >= 1), ms, -jnp.inf)
        mx = jnp.max(cand)
        b = jnp.min(jnp.where(cand == mx, idx, jnp.int32(1 << 20)))
        cb = jnp.sum(jnp.where(idx == b, codes, 0))
        yb = cb >> 12
        xb = (cb >> 3) & 511
        lb = cb & 7
        h = jnp.left_shift(jnp.int32(8), lb)
        lc = lb - 1
        base = 1 + 4 * t
        ac = jnp.where(idx == b, 0, ac)
        cys = (yb, yb, yb + h, yb + h)
        cxs = (xb, xb + h, xb, xb + h)
        ri = jnp.right_shift(yb, lc + 4)
        ci = jnp.right_shift(xb, lc + 4)
        row = 64 - jnp.right_shift(jnp.int32(64), lc) + ri
        rowa = jnp.right_shift(row, 3) * 8
        sr = row - rowa
        blk = ep_ref[pl.ds(rowa, 8), :]
        mks = []
        for k in range(4):
            mask = (sub8 == sr + (k >> 1)) & (lane8 == ci + (k & 1))
            mks.append(jnp.sum(jnp.where(mask, blk, 0.0)))
        for k in range(4):
            sel = idx == (base + k)
            codes = jnp.where(sel, (cys[k] << 12) | (cxs[k] << 3) | lc, codes)
            ms = jnp.where(sel, mks[k], ms)
            ac = jnp.where(sel, 1, ac)
        return codes, ms, ac

    codes, ms, ac = lax.fori_loop(0, _NSPLIT, split_body, (codes0, ms0, ac0))

    y5 = codes >> 16
    x5 = (codes >> 7) & 31
    key = jnp.zeros((1, _PAD), _I32)
    for b in range(5):
        key = key | (((y5 >> b) & 1) << (2 * b + 1)) | (((x5 >> b) & 1) << (2 * b))

    sq_i = lax.broadcasted_iota(_I32, (_PAD, _PAD), 0)
    sq_j = lax.broadcasted_iota(_I32, (_PAD, _PAD), 1)
    diag = (sq_i == sq_j)
    ones_col = jnp.full((_PAD, 1), 1.0, _F32)

    def _transp(v):
        vb = jnp.broadcast_to(v.astype(_F32), (_PAD, _PAD))
        return _dot(jnp.where(diag, vb, 0.0), ones_col)

    keyf = key.astype(_F32)
    actf = (ac == 1)
    key_s = jnp.broadcast_to(_transp(key), (_PAD, _PAD))
    key_l = jnp.broadcast_to(keyf, (_PAD, _PAD))
    act_l = jnp.broadcast_to(actf, (_PAD, _PAD))
    cnt = jnp.where((key_l < key_s) & act_l, 1.0, 0.0)
    rank_col = jnp.sum(cnt, axis=1, keepdims=True)
    rank_s = jnp.broadcast_to(rank_col, (_PAD, _PAD))
    rank_l = _dot(jnp.full((1, _PAD), 1.0, _F32),
                  jnp.where(diag, rank_s, 0.0))
    pm = jnp.where((sq_i.astype(_F32) == jnp.broadcast_to(rank_l, (_PAD, _PAD)))
                   & act_l, 1.0, 0.0)
    ordered = _dot(pm, _transp(codes))

    subcol = lax.broadcasted_iota(_I32, (_PAD, 1), 0)

    def gather_one(p):
        code = jnp.sum(jnp.where(subcol == p, ordered, 0.0)).astype(_I32)
        yb = code >> 12
        xb = (code >> 3) & 511
        lb = code & 7
        row = (1024 - jnp.right_shift(jnp.int32(1024), lb)
               + jnp.right_shift(yb, lb))
        row = jnp.right_shift(row, 3) * 8
        colv = jnp.right_shift(xb, lb)
        rows16 = ip_ref[:, pl.ds(row, 16), :]
        patch = pltpu.roll(rows16, -colv, 2)[:, :, 0:16]
        out_ref[:, pl.ds(p, 1), :, :] = patch[:, None, :, :]

    def gather_body(i, carry):
        gather_one(4 * i)
        gather_one(4 * i + 1)
        gather_one(4 * i + 2)
        gather_one(4 * i + 3)
        return carry

    lax.fori_loop(0, _FIXED // 4, gather_body, 0)


def kernel(img, edges):
    patches = pl.pallas_call(
        _kernel_body,
        out_shape=jax.ShapeDtypeStruct((3, _FIXED, 16, 16), _F32),
        scratch_shapes=[
            pltpu.VMEM((64, 32), _F32),
            pltpu.VMEM((3, 1024, 512), _F32),
        ],
    )(img, edges)
    return jnp.reshape(jnp.transpose(patches, (1, 2, 3, 0)), (3, _FIXED, 256))
